# Initial kernel scaffold; baseline (speedup 1.0000x reference)
import jax
import jax.numpy as jnp
from jax.experimental import pallas as pl


def kernel(x, W_fc, b_fc, edge_index_1, edge_vals_1, edge_index_2, edge_vals_2, W_gcn_1, b_gcn_1, prelu_1, W_gcn_2, b_gcn_2, prelu_2, W_att, b_att, att_vec):
    raise NotImplementedError("write your pallas kernel here")


# speedup vs baseline: 4.3130x; 4.3130x over previous
import jax
import jax.numpy as jnp
from jax import lax
from jax.experimental import pallas as pl
from jax.experimental.pallas import tpu as pltpu
from jax.experimental.pallas import tpu_sc as plsc

_N = 10000
_E = 320000
_D = 128
_DH = _D // 2

_NC = 2
_NS = 16
_EW = _E // _NS
_CH = 80
_NCHUNK = _EW // _CH
_RPT = _N // _NS
_ZR = 320
_LANES = 16

_BR = 1000


def _proj_body(x_ref, wfc_ref, bfc_ref, w1_ref, w2_ref,
               s1a_ref, s1b_ref, s2a_ref, s2b_ref):
    h = jnp.dot(x_ref[...], wfc_ref[...], preferred_element_type=jnp.float32)
    h = h + bfc_ref[...]
    h = jnp.where(h > 0, h, jnp.exp(h) - 1.0)
    s1 = jnp.dot(h, w1_ref[...], preferred_element_type=jnp.float32)
    s2 = jnp.dot(h, w2_ref[...], preferred_element_type=jnp.float32)
    s1a_ref[...] = s1[:, :_DH]
    s1b_ref[...] = s1[:, _DH:]
    s2a_ref[...] = s2[:, :_DH]
    s2b_ref[...] = s2[:, _DH:]


def _proj(x, wfc_t, bfc, w1_t, w2_t):
    return pl.pallas_call(
        _proj_body,
        grid=(_N // _BR,),
        in_specs=[
            pl.BlockSpec((_BR, _D), lambda i: (i, 0)),
            pl.BlockSpec((_D, _D), lambda i: (0, 0)),
            pl.BlockSpec((1, _D), lambda i: (0, 0)),
            pl.BlockSpec((_D, _D), lambda i: (0, 0)),
            pl.BlockSpec((_D, _D), lambda i: (0, 0)),
        ],
        out_specs=[pl.BlockSpec((_BR, _DH), lambda i: (i, 0))] * 4,
        out_shape=[jax.ShapeDtypeStruct((_N, _DH), jnp.float32)] * 4,
    )(x, wfc_t, bfc, w1_t, w2_t)


def _sc_body(s1a_hbm, s1b_hbm, src1_hbm, dst1_hbm, val1_hbm,
             s2a_hbm, s2b_hbm, src2_hbm, dst2_hbm, val2_hbm,
             o1_hbm, o2_hbm,
             srcv, dstv, valv, rows, zbuf, acc, sem):
    c = lax.axis_index("c")
    s = lax.axis_index("s")
    row0 = s * _RPT

    zero16 = jnp.zeros((_LANES,), jnp.float32)

    @pl.loop(0, _ZR)
    def _(i):
        for j in range(_DH // _LANES):
            zbuf[i, pl.ds(j * _LANES, _LANES)] = zero16

    def zero_acc():
        pltpu.sync_copy(zbuf, acc.at[pl.ds(row0, _ZR)])
        pltpu.sync_copy(zbuf.at[pl.ds(0, _RPT - _ZR)],
                        acc.at[pl.ds(row0 + _ZR, _RPT - _ZR)])

    def chunk_loop(s_hbm):
        @pl.loop(0, _NCHUNK)
        def _(k):
            pltpu.async_copy(s_hbm.at[srcv.at[k]], rows, sem).wait()

            for g in range(_CH // _LANES):
                vv = valv[k, pl.ds(g * _LANES, _LANES)]
                for l in range(_LANES):
                    e = g * _LANES + l
                    vb = jnp.full((_LANES,), vv[l], jnp.float32)
                    for j in range(_DH // _LANES):
                        sl = pl.ds(j * _LANES, _LANES)
                        rows[e, sl] = rows[e, sl] * vb

            pltpu.sync_copy(rows, acc.at[dstv.at[k]], add=True)

    def run_mp(sa_hbm, sb_hbm, src_hbm, dst_hbm, val_hbm, o_hbm):
        pltpu.sync_copy(src_hbm.at[s], srcv)
        pltpu.sync_copy(dst_hbm.at[s], dstv)
        pltpu.sync_copy(val_hbm.at[s], valv)
        plsc.subcore_barrier()

        @pl.when(c == 0)
        def _():
            chunk_loop(sa_hbm)

        @pl.when(c == 1)
        def _():
            chunk_loop(sb_hbm)

        plsc.subcore_barrier()
        pltpu.sync_copy(acc.at[pl.ds(row0, _RPT)],
                        o_hbm.at[c, pl.ds(row0, _RPT)])

    zero_acc()
    run_mp(s1a_hbm, s1b_hbm, src1_hbm, dst1_hbm, val1_hbm, o1_hbm)
    zero_acc()
    run_mp(s2a_hbm, s2b_hbm, src2_hbm, dst2_hbm, val2_hbm, o2_hbm)


def _sc_segsum(s1a, s1b, src1, dst1, val1, s2a, s2b, src2, dst2, val2):
    call = pl.kernel(
        _sc_body,
        out_type=(
            jax.ShapeDtypeStruct((_NC, _N, _DH), jnp.float32),
            jax.ShapeDtypeStruct((_NC, _N, _DH), jnp.float32),
        ),
        mesh=plsc.VectorSubcoreMesh(core_axis_name="c", subcore_axis_name="s"),
        compiler_params=pltpu.CompilerParams(use_tc_tiling_on_sc=False),
        scratch_types=[
            pltpu.VMEM((_NCHUNK, _CH), jnp.int32),
            pltpu.VMEM((_NCHUNK, _CH), jnp.int32),
            pltpu.VMEM((_NCHUNK, _CH), jnp.float32),
            pltpu.VMEM((_CH, _DH), jnp.float32),
            pltpu.VMEM((_ZR, _DH), jnp.float32),
            pltpu.VMEM_SHARED((_N, _DH), jnp.float32),
            pltpu.SemaphoreType.DMA,
        ],
    )
    return call(s1a, s1b, src1, dst1, val1, s2a, s2b, src2, dst2, val2)


def _post_body(o1_ref, o2_ref, b1_ref, a1_ref, b2_ref, a2_ref,
               watt_ref, batt_ref, e1_ref, e2_ref, sp_ref):
    a1 = a1_ref[0]
    a2 = a2_ref[0]
    e1 = jnp.concatenate([o1_ref[0], o1_ref[1]], axis=1) + b1_ref[...]
    e1 = jnp.where(e1 > 0, e1, a1 * e1)
    e1_ref[...] = e1
    e2 = jnp.concatenate([o2_ref[0], o2_ref[1]], axis=1) + b2_ref[...]
    e2 = jnp.where(e2 > 0, e2, a2 * e2)
    e2_ref[...] = e2
    t1 = jnp.tanh(jnp.dot(e1, watt_ref[...], preferred_element_type=jnp.float32)
                  + batt_ref[...])
    t2 = jnp.tanh(jnp.dot(e2, watt_ref[...], preferred_element_type=jnp.float32)
                  + batt_ref[...])
    upd = jnp.concatenate([jnp.sum(t1, axis=0, keepdims=True),
                           jnp.sum(t2, axis=0, keepdims=True)], axis=0)

    @pl.when(pl.program_id(0) == 0)
    def _():
        sp_ref[...] = upd

    @pl.when(pl.program_id(0) != 0)
    def _():
        sp_ref[...] = sp_ref[...] + upd


def _post(o1, o2, b1, a1, b2, a2, watt_t, batt):
    return pl.pallas_call(
        _post_body,
        grid=(_N // _BR,),
        in_specs=[
            pl.BlockSpec((_NC, _BR, _DH), lambda i: (0, i, 0)),
            pl.BlockSpec((_NC, _BR, _DH), lambda i: (0, i, 0)),
            pl.BlockSpec((1, _D), lambda i: (0, 0)),
            pl.BlockSpec(memory_space=pltpu.SMEM),
            pl.BlockSpec((1, _D), lambda i: (0, 0)),
            pl.BlockSpec(memory_space=pltpu.SMEM),
            pl.BlockSpec((_D, _D), lambda i: (0, 0)),
            pl.BlockSpec((1, _D), lambda i: (0, 0)),
        ],
        out_specs=[
            pl.BlockSpec((_BR, _D), lambda i: (i, 0)),
            pl.BlockSpec((_BR, _D), lambda i: (i, 0)),
            pl.BlockSpec((2, _D), lambda i: (0, 0)),
        ],
        out_shape=[
            jax.ShapeDtypeStruct((_N, _D), jnp.float32),
            jax.ShapeDtypeStruct((_N, _D), jnp.float32),
            jax.ShapeDtypeStruct((2, _D), jnp.float32),
        ],
    )(o1, o2, b1, a1, b2, a2, watt_t, batt)


def _combine_body(e1_ref, e2_ref, sp_ref, att_ref, z_ref):
    inv_n = 1.0 / _N
    l1 = jnp.sum(att_ref[...] * sp_ref[0:1, :]) * inv_n
    l2 = jnp.sum(att_ref[...] * sp_ref[1:2, :]) * inv_n
    m = jnp.maximum(l1, l2)
    w1 = jnp.exp(l1 - m)
    w2 = jnp.exp(l2 - m)
    denom = w1 + w2
    z_ref[...] = (w1 * e1_ref[...] + w2 * e2_ref[...]) / denom


def _combine(e1, e2, sp, att_vec):
    return pl.pallas_call(
        _combine_body,
        grid=(_N // _BR,),
        in_specs=[
            pl.BlockSpec((_BR, _D), lambda i: (i, 0)),
            pl.BlockSpec((_BR, _D), lambda i: (i, 0)),
            pl.BlockSpec((2, _D), lambda i: (0, 0)),
            pl.BlockSpec((1, _D), lambda i: (0, 0)),
        ],
        out_specs=pl.BlockSpec((_BR, _D), lambda i: (i, 0)),
        out_shape=jax.ShapeDtypeStruct((_N, _D), jnp.float32),
    )(e1, e2, sp, att_vec)


def kernel(x, W_fc, b_fc, edge_index_1, edge_vals_1, edge_index_2, edge_vals_2,
           W_gcn_1, b_gcn_1, prelu_1, W_gcn_2, b_gcn_2, prelu_2,
           W_att, b_att, att_vec):
    s1a, s1b, s2a, s2b = _proj(x, W_fc.T, b_fc.reshape(1, _D),
                               W_gcn_1.T, W_gcn_2.T)

    dst1 = edge_index_1[0].reshape(_NS, _NCHUNK, _CH)
    src1 = edge_index_1[1].reshape(_NS, _NCHUNK, _CH)
    val1 = edge_vals_1.reshape(_NS, _NCHUNK, _CH)
    dst2 = edge_index_2[0].reshape(_NS, _NCHUNK, _CH)
    src2 = edge_index_2[1].reshape(_NS, _NCHUNK, _CH)
    val2 = edge_vals_2.reshape(_NS, _NCHUNK, _CH)

    o1, o2 = _sc_segsum(s1a, s1b, src1, dst1, val1,
                        s2a, s2b, src2, dst2, val2)

    e1, e2, sp = _post(o1, o2, b_gcn_1.reshape(1, _D), prelu_1,
                       b_gcn_2.reshape(1, _D), prelu_2, W_att.T,
                       b_att.reshape(1, _D))
    return _combine(e1, e2, sp, att_vec)


# speedup vs baseline: 7.1548x; 1.6589x over previous
import jax
import jax.numpy as jnp
from jax import lax
from jax.experimental import pallas as pl
from jax.experimental.pallas import tpu as pltpu
from jax.experimental.pallas import tpu_sc as plsc

_N = 10000
_E = 320000
_D = 128
_DH = _D // 2

_NC = 2
_NS = 16
_EW = _E // _NS
_CH = 80
_NCHUNK = _EW // _CH
_RPT = _N // _NS
_ZR = 320
_LANES = 16

_BR = 1000


def _proj_body(x_ref, wfc_ref, bfc_ref, w1_ref, w2_ref,
               s1a_ref, s1b_ref, s2a_ref, s2b_ref):
    h = jnp.dot(x_ref[...], wfc_ref[...], preferred_element_type=jnp.float32)
    h = h + bfc_ref[...]
    h = jnp.where(h > 0, h, jnp.exp(h) - 1.0)
    s1 = jnp.dot(h, w1_ref[...], preferred_element_type=jnp.float32)
    s2 = jnp.dot(h, w2_ref[...], preferred_element_type=jnp.float32)
    s1a_ref[...] = s1[:, :_DH]
    s1b_ref[...] = s1[:, _DH:]
    s2a_ref[...] = s2[:, :_DH]
    s2b_ref[...] = s2[:, _DH:]


def _proj(x, wfc_t, bfc, w1_t, w2_t):
    return pl.pallas_call(
        _proj_body,
        grid=(_N // _BR,),
        in_specs=[
            pl.BlockSpec((_BR, _D), lambda i: (i, 0)),
            pl.BlockSpec((_D, _D), lambda i: (0, 0)),
            pl.BlockSpec((1, _D), lambda i: (0, 0)),
            pl.BlockSpec((_D, _D), lambda i: (0, 0)),
            pl.BlockSpec((_D, _D), lambda i: (0, 0)),
        ],
        out_specs=[pl.BlockSpec((_BR, _DH), lambda i: (i, 0))] * 4,
        out_shape=[jax.ShapeDtypeStruct((_N, _DH), jnp.float32)] * 4,
    )(x, wfc_t, bfc, w1_t, w2_t)


def _sc_body(s1a_hbm, s1b_hbm, src1_hbm, dst1_hbm, val1_hbm,
             s2a_hbm, s2b_hbm, src2_hbm, dst2_hbm, val2_hbm,
             o1_hbm, o2_hbm,
             srcv, dstv, valv, rows, rows2, zbuf, acc, sem, sem2):
    c = lax.axis_index("c")
    s = lax.axis_index("s")
    row0 = s * _RPT

    zero16 = jnp.zeros((_LANES,), jnp.float32)

    @pl.loop(0, _ZR)
    def _(i):
        for j in range(_DH // _LANES):
            zbuf[i, pl.ds(j * _LANES, _LANES)] = zero16

    def zero_acc():
        pltpu.sync_copy(zbuf, acc.at[pl.ds(row0, _ZR)])
        pltpu.sync_copy(zbuf.at[pl.ds(0, _RPT - _ZR)],
                        acc.at[pl.ds(row0 + _ZR, _RPT - _ZR)])

    def chunk_loop(s_hbm):
        def scale(k, buf):
            for g in range(_CH // _LANES):
                vv = valv[k, pl.ds(g * _LANES, _LANES)]
                for l in range(_LANES):
                    e = g * _LANES + l
                    vb = jnp.full((_LANES,), vv[l], jnp.float32)
                    for j in range(_DH // _LANES):
                        sl = pl.ds(j * _LANES, _LANES)
                        buf[e, sl] = buf[e, sl] * vb

        def half(k, buf, sem):
            pltpu.make_async_copy(s_hbm.at[srcv.at[k]], buf, sem).wait()
            scale(k, buf)
            pltpu.sync_copy(buf, acc.at[dstv.at[k]], add=True)

            @pl.when(k + 2 < _NCHUNK)
            def _():
                pltpu.async_copy(s_hbm.at[srcv.at[k + 2]], buf, sem)

        pltpu.async_copy(s_hbm.at[srcv.at[0]], rows, sem)
        pltpu.async_copy(s_hbm.at[srcv.at[1]], rows2, sem2)
        plsc.subcore_barrier()

        @pl.loop(0, _NCHUNK, step=2)
        def _(k):
            half(k, rows, sem)
            half(k + 1, rows2, sem2)

    def run_mp(sa_hbm, sb_hbm, src_hbm, dst_hbm, val_hbm, o_hbm):
        pltpu.sync_copy(src_hbm.at[s], srcv)
        pltpu.sync_copy(dst_hbm.at[s], dstv)
        pltpu.sync_copy(val_hbm.at[s], valv)

        @pl.when(c == 0)
        def _():
            chunk_loop(sa_hbm)

        @pl.when(c == 1)
        def _():
            chunk_loop(sb_hbm)

        plsc.subcore_barrier()
        pltpu.sync_copy(acc.at[pl.ds(row0, _RPT)],
                        o_hbm.at[c, pl.ds(row0, _RPT)])

    zero_acc()
    run_mp(s1a_hbm, s1b_hbm, src1_hbm, dst1_hbm, val1_hbm, o1_hbm)
    zero_acc()
    run_mp(s2a_hbm, s2b_hbm, src2_hbm, dst2_hbm, val2_hbm, o2_hbm)


def _sc_segsum(s1a, s1b, src1, dst1, val1, s2a, s2b, src2, dst2, val2):
    call = pl.kernel(
        _sc_body,
        out_type=(
            jax.ShapeDtypeStruct((_NC, _N, _DH), jnp.float32),
            jax.ShapeDtypeStruct((_NC, _N, _DH), jnp.float32),
        ),
        mesh=plsc.VectorSubcoreMesh(core_axis_name="c", subcore_axis_name="s"),
        compiler_params=pltpu.CompilerParams(use_tc_tiling_on_sc=False),
        scratch_types=[
            pltpu.VMEM((_NCHUNK, _CH), jnp.int32),
            pltpu.VMEM((_NCHUNK, _CH), jnp.int32),
            pltpu.VMEM((_NCHUNK, _CH), jnp.float32),
            pltpu.VMEM((_CH, _DH), jnp.float32),
            pltpu.VMEM((_CH, _DH), jnp.float32),
            pltpu.VMEM((_ZR, _DH), jnp.float32),
            pltpu.VMEM_SHARED((_N, _DH), jnp.float32),
            pltpu.SemaphoreType.DMA,
            pltpu.SemaphoreType.DMA,
        ],
    )
    return call(s1a, s1b, src1, dst1, val1, s2a, s2b, src2, dst2, val2)


def _post_body(o1_ref, o2_ref, b1_ref, a1_ref, b2_ref, a2_ref,
               watt_ref, batt_ref, e1_ref, e2_ref, sp_ref):
    a1 = a1_ref[0]
    a2 = a2_ref[0]
    e1 = jnp.concatenate([o1_ref[0], o1_ref[1]], axis=1) + b1_ref[...]
    e1 = jnp.where(e1 > 0, e1, a1 * e1)
    e1_ref[...] = e1
    e2 = jnp.concatenate([o2_ref[0], o2_ref[1]], axis=1) + b2_ref[...]
    e2 = jnp.where(e2 > 0, e2, a2 * e2)
    e2_ref[...] = e2
    t1 = jnp.tanh(jnp.dot(e1, watt_ref[...], preferred_element_type=jnp.float32)
                  + batt_ref[...])
    t2 = jnp.tanh(jnp.dot(e2, watt_ref[...], preferred_element_type=jnp.float32)
                  + batt_ref[...])
    upd = jnp.concatenate([jnp.sum(t1, axis=0, keepdims=True),
                           jnp.sum(t2, axis=0, keepdims=True)], axis=0)

    @pl.when(pl.program_id(0) == 0)
    def _():
        sp_ref[...] = upd

    @pl.when(pl.program_id(0) != 0)
    def _():
        sp_ref[...] = sp_ref[...] + upd


def _post(o1, o2, b1, a1, b2, a2, watt_t, batt):
    return pl.pallas_call(
        _post_body,
        grid=(_N // _BR,),
        in_specs=[
            pl.BlockSpec((_NC, _BR, _DH), lambda i: (0, i, 0)),
            pl.BlockSpec((_NC, _BR, _DH), lambda i: (0, i, 0)),
            pl.BlockSpec((1, _D), lambda i: (0, 0)),
            pl.BlockSpec(memory_space=pltpu.SMEM),
            pl.BlockSpec((1, _D), lambda i: (0, 0)),
            pl.BlockSpec(memory_space=pltpu.SMEM),
            pl.BlockSpec((_D, _D), lambda i: (0, 0)),
            pl.BlockSpec((1, _D), lambda i: (0, 0)),
        ],
        out_specs=[
            pl.BlockSpec((_BR, _D), lambda i: (i, 0)),
            pl.BlockSpec((_BR, _D), lambda i: (i, 0)),
            pl.BlockSpec((2, _D), lambda i: (0, 0)),
        ],
        out_shape=[
            jax.ShapeDtypeStruct((_N, _D), jnp.float32),
            jax.ShapeDtypeStruct((_N, _D), jnp.float32),
            jax.ShapeDtypeStruct((2, _D), jnp.float32),
        ],
    )(o1, o2, b1, a1, b2, a2, watt_t, batt)


def _combine_body(e1_ref, e2_ref, sp_ref, att_ref, z_ref):
    inv_n = 1.0 / _N
    l1 = jnp.sum(att_ref[...] * sp_ref[0:1, :]) * inv_n
    l2 = jnp.sum(att_ref[...] * sp_ref[1:2, :]) * inv_n
    m = jnp.maximum(l1, l2)
    w1 = jnp.exp(l1 - m)
    w2 = jnp.exp(l2 - m)
    denom = w1 + w2
    z_ref[...] = (w1 * e1_ref[...] + w2 * e2_ref[...]) / denom


def _combine(e1, e2, sp, att_vec):
    return pl.pallas_call(
        _combine_body,
        grid=(_N // _BR,),
        in_specs=[
            pl.BlockSpec((_BR, _D), lambda i: (i, 0)),
            pl.BlockSpec((_BR, _D), lambda i: (i, 0)),
            pl.BlockSpec((2, _D), lambda i: (0, 0)),
            pl.BlockSpec((1, _D), lambda i: (0, 0)),
        ],
        out_specs=pl.BlockSpec((_BR, _D), lambda i: (i, 0)),
        out_shape=jax.ShapeDtypeStruct((_N, _D), jnp.float32),
    )(e1, e2, sp, att_vec)


def kernel(x, W_fc, b_fc, edge_index_1, edge_vals_1, edge_index_2, edge_vals_2,
           W_gcn_1, b_gcn_1, prelu_1, W_gcn_2, b_gcn_2, prelu_2,
           W_att, b_att, att_vec):
    s1a, s1b, s2a, s2b = _proj(x, W_fc.T, b_fc.reshape(1, _D),
                               W_gcn_1.T, W_gcn_2.T)

    dst1 = edge_index_1[0].reshape(_NS, _NCHUNK, _CH)
    src1 = edge_index_1[1].reshape(_NS, _NCHUNK, _CH)
    val1 = edge_vals_1.reshape(_NS, _NCHUNK, _CH)
    dst2 = edge_index_2[0].reshape(_NS, _NCHUNK, _CH)
    src2 = edge_index_2[1].reshape(_NS, _NCHUNK, _CH)
    val2 = edge_vals_2.reshape(_NS, _NCHUNK, _CH)

    o1, o2 = _sc_segsum(s1a, s1b, src1, dst1, val1,
                        s2a, s2b, src2, dst2, val2)

    e1, e2, sp = _post(o1, o2, b_gcn_1.reshape(1, _D), prelu_1,
                       b_gcn_2.reshape(1, _D), prelu_2, W_att.T,
                       b_att.reshape(1, _D))
    return _combine(e1, e2, sp, att_vec)
